# baseline (speedup 1.0000x reference)
import functools

import jax
import jax.numpy as jnp
from jax import lax
from jax.experimental import pallas as pl
from jax.experimental.pallas import tpu as pltpu
from jax.experimental.pallas import tpu_sc as plsc

_B = 4096
_D = 1024
_NC = 2
_NS = 16
_NW = _NC * _NS
_L = 16
_BPW = _B // _NW
_CH = 16
_NCHUNK = _BPW // _CH


_NBUF = 3


@functools.partial(
    pl.kernel,
    out_type=jax.ShapeDtypeStruct((_NW, _L), jnp.float32),
    mesh=plsc.VectorSubcoreMesh(core_axis_name="c", subcore_axis_name="s"),
    scratch_types=[
        pltpu.VMEM((_BPW,), jnp.int32),
        pltpu.VMEM((_NBUF, _CH, _D), jnp.float32),
        pltpu.VMEM((_NBUF, _CH, _D), jnp.float32),
        pltpu.VMEM((_L,), jnp.float32),
        pltpu.SemaphoreType.DMA,
        pltpu.SemaphoreType.DMA,
        pltpu.SemaphoreType.DMA,
        pltpu.SemaphoreType.DMA,
        pltpu.SemaphoreType.DMA,
        pltpu.SemaphoreType.DMA,
    ],
)
def _center_loss_partials(x_hbm, y_hbm, tab_hbm, out_hbm,
                          idx_v, xbuf, cbuf, accv,
                          sx0, sx1, sx2, sc0, sc1, sc2):
    wid = lax.axis_index("s") * _NC + lax.axis_index("c")
    base = wid * _BPW
    pltpu.sync_copy(y_hbm.at[pl.ds(base, _BPW)], idx_v)

    xsems, csems = (sx0, sx1, sx2), (sc0, sc1, sc2)

    def start(ch):
        b = ch % _NBUF
        row0 = base + ch * _CH
        cpx = pltpu.async_copy(x_hbm.at[pl.ds(row0, _CH)], xbuf.at[b],
                               xsems[b])
        cpc = pltpu.async_copy(tab_hbm.at[idx_v.at[pl.ds(ch * _CH, _CH)]],
                               cbuf.at[b], csems[b])
        return cpx, cpc

    inflight = [None] * _NBUF
    for ch in range(_NBUF - 1):
        inflight[ch] = start(ch)

    acc = jnp.zeros((_L,), jnp.float32)
    for ch in range(_NCHUNK):
        b = ch % _NBUF
        nxt = ch + _NBUF - 1
        if nxt < _NCHUNK:
            inflight[nxt % _NBUF] = start(nxt)
        cpx, cpc = inflight[b]
        cpx.wait()
        cpc.wait()
        xb, cb = xbuf.at[b], cbuf.at[b]

        def body(r, a):
            p = [None] * 4
            for j in range(_D // _L):
                xv = xb[r, pl.ds(j * _L, _L)]
                cv = cb[r, pl.ds(j * _L, _L)]
                dv = xv - cv
                dd = dv * dv
                p[j % 4] = dd if p[j % 4] is None else p[j % 4] + dd
            return a + ((p[0] + p[1]) + (p[2] + p[3]))

        acc = lax.fori_loop(0, _CH, body, acc)
    accv[...] = acc
    pltpu.sync_copy(accv, out_hbm.at[wid])


def kernel(output_features, y_truth, feature_centers):
    batch = y_truth.shape[0]
    x = output_features.reshape(batch, -1)
    partials = _center_loss_partials(
        x, y_truth.astype(jnp.int32), feature_centers)
    return (0.5 / batch) * jnp.sum(partials)


# speedup vs baseline: 1.7031x; 1.7031x over previous
import functools

import jax
import jax.numpy as jnp
from jax import lax
from jax.experimental import pallas as pl
from jax.experimental.pallas import tpu as pltpu
from jax.experimental.pallas import tpu_sc as plsc

_B = 4096
_D = 1024
_NC = 2
_NS = 16
_NW = _NC * _NS
_L = 16
_BPW = _B // _NW
_CH = 16
_NCHUNK = _BPW // _CH
_UNROLL = 8


@functools.partial(
    pl.kernel,
    out_type=jax.ShapeDtypeStruct((_NW, _L), jnp.float32),
    mesh=plsc.VectorSubcoreMesh(core_axis_name="c", subcore_axis_name="s"),
    scratch_types=[
        pltpu.VMEM((_BPW,), jnp.int32),
        pltpu.VMEM((_CH, _D), jnp.float32),
        pltpu.VMEM((_CH, _D), jnp.float32),
        pltpu.VMEM((_CH, _D), jnp.float32),
        pltpu.VMEM((_CH, _D), jnp.float32),
        pltpu.VMEM((_L,), jnp.float32),
        pltpu.SemaphoreType.DMA,
        pltpu.SemaphoreType.DMA,
        pltpu.SemaphoreType.DMA,
        pltpu.SemaphoreType.DMA,
    ],
)
def _center_loss_partials(x_hbm, y_hbm, tab_hbm, out_hbm,
                          idx_v, xb0, cb0, xb1, cb1, accv,
                          sx0, sc0, sx1, sc1):
    wid = lax.axis_index("s") * _NC + lax.axis_index("c")
    base = wid * _BPW
    pltpu.sync_copy(y_hbm.at[pl.ds(base, _BPW)], idx_v)

    def start(ch, xb, cb, sx, sc):
        row0 = base + ch * _CH
        pltpu.async_copy(x_hbm.at[pl.ds(row0, _CH)], xb, sx)
        pltpu.async_copy(tab_hbm.at[idx_v.at[pl.ds(ch * _CH, _CH)]], cb, sc)

    def wait(xb, cb, sx, sc):
        pltpu.make_async_copy(x_hbm.at[pl.ds(0, _CH)], xb, sx).wait()
        pltpu.make_async_copy(tab_hbm.at[pl.ds(0, _CH)], cb, sc).wait()

    def compute(xb, cb, acc):
        def row_body(r, a):
            def jj_body(jj, a2):
                col = jj * (_UNROLL * _L)
                p0 = None
                p1 = None
                for k in range(_UNROLL):
                    xv = xb[r, pl.ds(col + k * _L, _L)]
                    cv = cb[r, pl.ds(col + k * _L, _L)]
                    dv = xv - cv
                    dd = dv * dv
                    if k % 2 == 0:
                        p0 = dd if p0 is None else p0 + dd
                    else:
                        p1 = dd if p1 is None else p1 + dd
                return a2 + (p0 + p1)

            return lax.fori_loop(0, _D // (_UNROLL * _L), jj_body, a)

        return lax.fori_loop(0, _CH, row_body, acc)

    start(0, xb0, cb0, sx0, sc0)
    start(1, xb1, cb1, sx1, sc1)

    def pair_body(t, acc):
        ch0 = 2 * t
        wait(xb0, cb0, sx0, sc0)
        acc = compute(xb0, cb0, acc)

        @pl.when(ch0 + 2 < _NCHUNK)
        def _():
            start(ch0 + 2, xb0, cb0, sx0, sc0)

        wait(xb1, cb1, sx1, sc1)
        acc = compute(xb1, cb1, acc)

        @pl.when(ch0 + 3 < _NCHUNK)
        def _():
            start(ch0 + 3, xb1, cb1, sx1, sc1)

        return acc

    acc = lax.fori_loop(0, _NCHUNK // 2, pair_body,
                        jnp.zeros((_L,), jnp.float32))
    accv[...] = acc
    pltpu.sync_copy(accv, out_hbm.at[wid])


def kernel(output_features, y_truth, feature_centers):
    batch = y_truth.shape[0]
    x = output_features.reshape(batch, -1)
    partials = _center_loss_partials(
        x, y_truth.astype(jnp.int32), feature_centers)
    return (0.5 / batch) * jnp.sum(partials)
